# baseline (speedup 1.0000x reference)
import jax
import jax.numpy as jnp
from jax.experimental import pallas as pl

_PREC = jax.lax.Precision.DEFAULT
_DN = (((1,), (0,)), ((), ()))
_B = 4
_K = 320


def _mm(a, b):
    return jax.lax.dot_general(a, b, _DN, precision=_PREC,
                               preferred_element_type=jnp.float32)


def _prep_w(wr, k):
    t = jnp.transpose(wr)
    t = t.reshape(512, k, 512)
    return jnp.transpose(t, (1, 0, 2)).reshape(512 * k, 512)


def _body(wav_ref, w0_ref, w1_ref, w2_ref, w3_ref, w4_ref,
          b0_ref, b1_ref, b2_ref, b3_ref, b4_ref, cb_ref, out_ref):
    w0 = jnp.transpose(w0_ref[...])
    w2 = _prep_w(w2_ref[...], 4)
    w3 = w3_ref
    w4 = w4_ref
    cts = [cb_ref[0], cb_ref[1]]
    c2s = [jnp.sum(ct * ct, axis=0, keepdims=True) for ct in cts]

    for b in range(_B):
        x = wav_ref[b]
        xx = jnp.concatenate([x[0:4799], x[1:4800]], axis=1)
        h = _mm(xx, w0)
        h = jnp.maximum(h + b0_ref[...], 0.0)

        f = h[0:4796].reshape(1199, 2048)
        h = _mm(f[0:1198], w1_ref[0:2048]) + _mm(f[1:1199], w1_ref[2048:4096])
        h = jnp.maximum(h + b1_ref[...], 0.0)

        f = h.reshape(599, 1024)
        h = _mm(f[0:598], w2[0:1024]) + _mm(f[1:599], w2[1024:2048])
        h = jnp.maximum(h + b2_ref[...], 0.0)

        f = h.reshape(299, 1024)
        h = _mm(f[0:298], w3[0:1024]) + _mm(f[1:299], w3[1024:2048])
        h = jnp.maximum(h + b3_ref[...], 0.0)

        f = h.reshape(149, 1024)
        h = _mm(f[0:148], w4[0:1024]) + _mm(f[1:149], w4[1024:2048])
        h = jnp.maximum(h + b4_ref[...], 0.0)

        for g in range(2):
            xg = h[:, 256 * g:256 * (g + 1)]
            x2 = jnp.sum(xg * xg, axis=1, keepdims=True)
            dist = (x2 - 2.0 * _mm(xg, cts[g])) + c2s[g]
            m = jnp.min(dist, axis=1, keepdims=True)
            k_iota = jax.lax.broadcasted_iota(jnp.int32, dist.shape, 1)
            idx = jnp.min(jnp.where(dist == m, k_iota, jnp.int32(_K)), axis=1)
            out_ref[b, g] = idx


def kernel(wav_input, conv_w0, conv_b0, conv_w1, conv_b1, conv_w2, conv_b2,
           conv_w3, conv_b3, conv_w4, conv_b4, codebook):
    wavf = wav_input.reshape(_B, 4800, 5)
    out = pl.pallas_call(
        _body,
        out_shape=jax.ShapeDtypeStruct((_B, 2, 148), jnp.int32),
    )(wavf, conv_w0.reshape(512, 10),
      conv_w1.transpose(2, 1, 0).reshape(4096, 512),
      conv_w2.reshape(512, 2048),
      conv_w3.transpose(2, 1, 0).reshape(2048, 512),
      conv_w4.transpose(2, 1, 0).reshape(2048, 512),
      conv_b0.reshape(1, 512), conv_b1.reshape(1, 512),
      conv_b2.reshape(1, 512), conv_b3.reshape(1, 512),
      conv_b4.reshape(1, 512), codebook.transpose(0, 2, 1))
    return out.transpose(0, 2, 1).reshape(_B, 296)


# speedup vs baseline: 1.1069x; 1.1069x over previous
import jax
import jax.numpy as jnp
from jax.experimental import pallas as pl

_PREC = jax.lax.Precision.DEFAULT
_DN = (((1,), (0,)), ((), ()))
_B = 4
_K = 320


def _mm(a, b):
    return jax.lax.dot_general(a, b, _DN, precision=_PREC,
                               preferred_element_type=jnp.float32)


def _body(wav_ref, w0_ref, w1_ref, w2_ref, w3_ref, w4_ref,
          b0_ref, b1_ref, b2_ref, b3_ref, b4_ref, ct_ref, out_ref):
    x = wav_ref[0]
    xx = jnp.concatenate([x[0:4799], x[1:4800]], axis=1)
    h = _mm(xx, w0_ref[...])
    h = jnp.maximum(h + b0_ref[...], 0.0)

    f = h[0:4796].reshape(1199, 2048)
    h = _mm(f[0:1198], w1_ref[0:2048]) + _mm(f[1:1199], w1_ref[2048:4096])
    h = jnp.maximum(h + b1_ref[...], 0.0)

    f = h.reshape(599, 1024)
    h = _mm(f[0:598], w2_ref[0:1024]) + _mm(f[1:599], w2_ref[1024:2048])
    h = jnp.maximum(h + b2_ref[...], 0.0)

    f = h.reshape(299, 1024)
    h = _mm(f[0:298], w3_ref[0:1024]) + _mm(f[1:299], w3_ref[1024:2048])
    h = jnp.maximum(h + b3_ref[...], 0.0)

    f = h.reshape(149, 1024)
    h = _mm(f[0:148], w4_ref[0:1024]) + _mm(f[1:149], w4_ref[1024:2048])
    h = jnp.maximum(h + b4_ref[...], 0.0)

    for g in range(2):
        xg = h[:, 256 * g:256 * (g + 1)]
        ct = ct_ref[g]
        x2 = jnp.sum(xg * xg, axis=1, keepdims=True)
        c2 = jnp.sum(ct * ct, axis=0, keepdims=True)
        dist = (x2 - 2.0 * _mm(xg, ct)) + c2
        m = jnp.min(dist, axis=1, keepdims=True)
        k_iota = jax.lax.broadcasted_iota(jnp.int32, dist.shape, 1)
        idx = jnp.min(jnp.where(dist == m, k_iota, jnp.int32(_K)), axis=1)
        out_ref[0, g] = idx


def kernel(wav_input, conv_w0, conv_b0, conv_w1, conv_b1, conv_w2, conv_b2,
           conv_w3, conv_b3, conv_w4, conv_b4, codebook):
    wavf = wav_input.reshape(_B, 4800, 5)
    w0 = conv_w0[:, 0, :].T
    w1 = conv_w1.transpose(2, 1, 0).reshape(4096, 512)
    w2 = conv_w2.transpose(2, 1, 0).reshape(2048, 512)
    w3 = conv_w3.transpose(2, 1, 0).reshape(2048, 512)
    w4 = conv_w4.transpose(2, 1, 0).reshape(2048, 512)
    ct = codebook.transpose(0, 2, 1)
    bs = [b.reshape(1, 512) for b in
          (conv_b0, conv_b1, conv_b2, conv_b3, conv_b4)]
    c = lambda b: (0, 0)
    out = pl.pallas_call(
        _body,
        grid=(_B,),
        in_specs=[
            pl.BlockSpec((1, 4800, 5), lambda b: (b, 0, 0)),
            pl.BlockSpec((10, 512), c), pl.BlockSpec((4096, 512), c),
            pl.BlockSpec((2048, 512), c), pl.BlockSpec((2048, 512), c),
            pl.BlockSpec((2048, 512), c),
            pl.BlockSpec((1, 512), c), pl.BlockSpec((1, 512), c),
            pl.BlockSpec((1, 512), c), pl.BlockSpec((1, 512), c),
            pl.BlockSpec((1, 512), c),
            pl.BlockSpec((2, 256, 320), lambda b: (0, 0, 0)),
        ],
        out_specs=pl.BlockSpec((1, 2, 148), lambda b: (b, 0, 0)),
        out_shape=jax.ShapeDtypeStruct((_B, 2, 148), jnp.int32),
    )(wavf, w0, w1, w2, w3, w4, *bs, ct)
    return out.transpose(0, 2, 1).reshape(_B, 296)


# speedup vs baseline: 1.3648x; 1.2329x over previous
import jax
import jax.numpy as jnp
from jax.experimental import pallas as pl

_PREC = jax.lax.Precision.DEFAULT
_DN = (((1,), (0,)), ((), ()))
_B = 4
_K = 320


def _mm(a, b):
    return jax.lax.dot_general(a, b, _DN, precision=_PREC,
                               preferred_element_type=jnp.float32)


def _body(wav_ref, w0_ref, w1_ref, w2_ref, w3_ref, w4_ref,
          b0_ref, b1_ref, b2_ref, b3_ref, b4_ref, ct_ref, out_ref):
    x = wav_ref[0]
    nxt = jnp.concatenate(
        [x[1:1200, 0:10], jnp.zeros((1, 10), jnp.float32)], axis=0)
    xw = jnp.concatenate([x, nxt], axis=1)
    f = jnp.maximum(_mm(xw, w0_ref[...]) + b0_ref[...], 0.0)

    h = _mm(f[0:1198], w1_ref[0:2048]) + _mm(f[1:1199], w1_ref[2048:4096])
    h = jnp.maximum(h + b1_ref[...], 0.0)

    f = h.reshape(599, 1024)
    h = _mm(f[0:598], w2_ref[0:1024]) + _mm(f[1:599], w2_ref[1024:2048])
    h = jnp.maximum(h + b2_ref[...], 0.0)

    f = h.reshape(299, 1024)
    h = _mm(f[0:298], w3_ref[0:1024]) + _mm(f[1:299], w3_ref[1024:2048])
    h = jnp.maximum(h + b3_ref[...], 0.0)

    f = h.reshape(149, 1024)
    h = _mm(f[0:148], w4_ref[0:1024]) + _mm(f[1:149], w4_ref[1024:2048])
    h = jnp.maximum(h + b4_ref[...], 0.0)

    for g in range(2):
        xg = h[:, 256 * g:256 * (g + 1)]
        ct = ct_ref[g]
        x2 = jnp.sum(xg * xg, axis=1, keepdims=True)
        c2 = jnp.sum(ct * ct, axis=0, keepdims=True)
        dist = (x2 - 2.0 * _mm(xg, ct)) + c2
        m = jnp.min(dist, axis=1, keepdims=True)
        k_iota = jax.lax.broadcasted_iota(jnp.int32, dist.shape, 1)
        idx = jnp.min(jnp.where(dist == m, k_iota, jnp.int32(_K)), axis=1)
        out_ref[0, g] = idx


def kernel(wav_input, conv_w0, conv_b0, conv_w1, conv_b1, conv_w2, conv_b2,
           conv_w3, conv_b3, conv_w4, conv_b4, codebook):
    wavf = wav_input.reshape(_B, 1200, 20)
    w0t = conv_w0[:, 0, :].T
    w0 = jnp.concatenate(
        [jnp.pad(w0t, ((5 * jj, 20 - 5 * jj), (0, 0))) for jj in range(4)],
        axis=1)
    b0 = jnp.tile(conv_b0, 4).reshape(1, 2048)
    w1 = conv_w1.transpose(2, 1, 0).reshape(4096, 512)
    w2 = conv_w2.transpose(2, 1, 0).reshape(2048, 512)
    w3 = conv_w3.transpose(2, 1, 0).reshape(2048, 512)
    w4 = conv_w4.transpose(2, 1, 0).reshape(2048, 512)
    ct = codebook.transpose(0, 2, 1)
    bs = [b.reshape(1, 512) for b in (conv_b1, conv_b2, conv_b3, conv_b4)]
    c = lambda b: (0, 0)
    out = pl.pallas_call(
        _body,
        grid=(_B,),
        in_specs=[
            pl.BlockSpec((1, 1200, 20), lambda b: (b, 0, 0)),
            pl.BlockSpec((30, 2048), c), pl.BlockSpec((4096, 512), c),
            pl.BlockSpec((2048, 512), c), pl.BlockSpec((2048, 512), c),
            pl.BlockSpec((2048, 512), c),
            pl.BlockSpec((1, 2048), c), pl.BlockSpec((1, 512), c),
            pl.BlockSpec((1, 512), c), pl.BlockSpec((1, 512), c),
            pl.BlockSpec((1, 512), c),
            pl.BlockSpec((2, 256, 320), lambda b: (0, 0, 0)),
        ],
        out_specs=pl.BlockSpec((1, 2, 148), lambda b: (b, 0, 0)),
        out_shape=jax.ShapeDtypeStruct((_B, 2, 148), jnp.int32),
    )(wavf, w0, w1, w2, w3, w4, b0, *bs, ct)
    return out.transpose(0, 2, 1).reshape(_B, 296)
